# Initial kernel scaffold; baseline (speedup 1.0000x reference)
import jax
import jax.numpy as jnp
from jax.experimental import pallas as pl


def kernel(x, edge_index, Wq, bq, Wk, bk, Wv, bv, Wo, bo, Wp, bp, gamma, beta):
    raise NotImplementedError("write your pallas kernel here")


# speedup vs baseline: 5.1854x; 5.1854x over previous
import jax
import jax.numpy as jnp
from jax.experimental import pallas as pl

N = 10000
D = 256
OUT = 256


def _body(x_ref, wv_ref, wo_ref, wp_ref, gamma_ref, beta_ref, o_ref):
    m_inner = jnp.dot(wo_ref[...], wv_ref[...], preferred_element_type=jnp.float32)
    m = jnp.dot(wp_ref[...], m_inner, preferred_element_type=jnp.float32)
    x = x_ref[...]
    z = jax.lax.dot_general(
        x, m, (((1,), (1,)), ((), ())), preferred_element_type=jnp.float32
    )
    zm = jnp.mean(z, axis=0, keepdims=True)
    zc = z - zm
    var = jnp.mean(zc * zc, axis=0, keepdims=True)
    scale = gamma_ref[...] * jax.lax.rsqrt(var + 1e-5)
    o_ref[...] = zc * scale + beta_ref[...]


def kernel(x, edge_index, Wq, bq, Wk, bk, Wv, bv, Wo, bo, Wp, bp, gamma, beta):
    del edge_index, Wq, bq, Wk, bk, bv, bo, bp
    out = pl.pallas_call(
        _body,
        out_shape=jax.ShapeDtypeStruct((N, OUT), jnp.float32),
    )(x, Wv, Wo, Wp, gamma.reshape(1, OUT), beta.reshape(1, OUT))
    return out
